# Initial kernel scaffold; baseline (speedup 1.0000x reference)
import jax
import jax.numpy as jnp
from jax.experimental import pallas as pl


def kernel(x, weight, lora_a, lora_b):
    raise NotImplementedError("write your pallas kernel here")


# speedup vs baseline: 4.6691x; 4.6691x over previous
import functools

import jax
import jax.numpy as jnp
from jax import lax
from jax.experimental import pallas as pl
from jax.experimental.pallas import tpu as pltpu
from jax.experimental.pallas import tpu_sc as plsc

_NC = 2
_NS = 16
_NW = _NC * _NS
_D = 64
_R = 16
_SCALE = 2.0
_CHUNK = 128


def _transpose_tc(a):
  r, v = a.shape
  blk = 2048
  grid = (v + blk - 1) // blk

  def body(a_ref, o_ref):
    o_ref[...] = a_ref[...].T

  return pl.pallas_call(
      body,
      grid=(grid,),
      in_specs=[pl.BlockSpec((r, blk), lambda j: (0, j))],
      out_specs=pl.BlockSpec((blk, r), lambda j: (j, 0)),
      out_shape=jax.ShapeDtypeStruct((v, r), a.dtype),
  )(a)


def _sc_body(x_hbm, w_hbm, at_hbm, bt_hbm, out_hbm,
             idx_v, w_rows, a_rows, bt_v, sem_w, sem_a):
  wid = lax.axis_index("s") * _NC + lax.axis_index("c")
  n_w = x_hbm.shape[0] // _NW
  n_chunks = n_w // _CHUNK

  pltpu.sync_copy(bt_hbm, bt_v)
  bt_vecs = [[bt_v[r, 16 * d:16 * (d + 1)] for d in range(4)]
             for r in range(_R)]

  def chunk(g, carry):
    base = wid * n_w + g * _CHUNK
    pltpu.sync_copy(x_hbm.at[pl.ds(base, _CHUNK)], idx_v)
    cp_w = pltpu.async_copy(w_hbm.at[idx_v], w_rows, sem_w)
    cp_a = pltpu.async_copy(at_hbm.at[idx_v], a_rows, sem_a)
    cp_w.wait()
    cp_a.wait()

    def per_index(i, c2):
      accs = [w_rows[i, 16 * d:16 * (d + 1)] for d in range(4)]
      a_vec = a_rows[i, :]
      for r in range(_R):
        s = a_vec[r]
        for d in range(4):
          accs[d] = accs[d] + bt_vecs[r][d] * s
      for d in range(4):
        w_rows[i, 16 * d:16 * (d + 1)] = accs[d]
      return c2

    lax.fori_loop(0, _CHUNK, per_index, 0)
    pltpu.sync_copy(w_rows, out_hbm.at[pl.ds(base, _CHUNK)])
    return carry

  lax.fori_loop(0, n_chunks, chunk, 0)


def kernel(x, weight, lora_a, lora_b):
  b, l = x.shape
  n = b * l
  xf = x.reshape(n)
  bt = (lora_b * _SCALE).T
  a_t = _transpose_tc(lora_a)

  mesh = plsc.VectorSubcoreMesh(core_axis_name="c", subcore_axis_name="s")
  f = pl.kernel(
      _sc_body,
      mesh=mesh,
      out_type=jax.ShapeDtypeStruct((n, _D), jnp.float32),
      scratch_types=[
          pltpu.VMEM((_CHUNK,), jnp.int32),
          pltpu.VMEM((_CHUNK, _D), jnp.float32),
          pltpu.VMEM((_CHUNK, _R), jnp.float32),
          pltpu.VMEM((_R, _D), jnp.float32),
          pltpu.SemaphoreType.DMA,
          pltpu.SemaphoreType.DMA,
      ],
      compiler_params=pltpu.CompilerParams(use_tc_tiling_on_sc=False),
  )
  out = f(xf, weight, a_t, bt)
  return out.reshape(b, l, _D)


# speedup vs baseline: 5.4954x; 1.1770x over previous
import jax
import jax.numpy as jnp
from jax import lax
from jax.experimental import pallas as pl
from jax.experimental.pallas import tpu as pltpu
from jax.experimental.pallas import tpu_sc as plsc

_NC = 2
_NS = 16
_NW = _NC * _NS
_D = 64
_R = 16
_SCALE = 2.0
_CHUNK = 128
_TW = 250


def _sc_body(x_hbm, w_hbm, a_hbm, bt_hbm, out_hbm, at_hbm,
             idx_all, w_rows0, w_rows1, a_rows0, a_rows1, bt_v,
             tbuf0, tbuf1, tout0, tout1,
             sem_w0, sem_w1, sem_a0, sem_a1, sem_o0, sem_o1,
             sem_ti0, sem_ti1, sem_to0, sem_to1):
  cid = lax.axis_index("c")
  tid = lax.axis_index("s")
  wid = tid * _NC + cid
  kt = a_hbm.shape[1] // _NS
  j0 = tid * kt
  g_total = idx_all.shape[0]
  n_w = g_total * _CHUNK
  lane_iota = lax.iota(jnp.int32, 16)

  tb = [tbuf0, tbuf1]
  to = [tout0, tout1]
  sti = [sem_ti0, sem_ti1]
  sto = [sem_to0, sem_to1]

  def ti_refs(k, s):
    return (a_hbm.at[:, j0 + k, :], tb[s], sti[s])

  def to_refs(k, s):
    return (to[s], at_hbm.at[cid, pl.ds((j0 + k) * _TW, _TW), :], sto[s])

  def ti_start(k, s):
    pltpu.async_copy(*ti_refs(k, s))

  def ti_wait(k, s):
    pltpu.make_async_copy(*ti_refs(k, s)).wait()

  def to_start(k, s):
    pltpu.async_copy(*to_refs(k, s))

  def to_wait(k, s):
    pltpu.make_async_copy(*to_refs(k, s)).wait()

  def t_cmp(s):
    tbuf, tout = tb[s], to[s]

    def body(i, c):
      vec = plsc.load_gather(tbuf, [lane_iota, jnp.full((16,), i, jnp.int32)])
      tout[i, :] = vec
      return c
    lax.fori_loop(0, _TW, body, 0)

  ti_start(0, 0)
  ti_start(1, 1)
  ti_wait(0, 0)
  t_cmp(0)
  to_start(0, 0)
  ti_start(2, 0)
  ti_wait(1, 1)
  t_cmp(1)
  to_start(1, 1)
  ti_start(3, 1)

  def t_pair(h, c):
    for off, s in ((2, 0), (3, 1)):
      k = 2 * h + off
      ti_wait(k, s)
      to_wait(k, s)
      t_cmp(s)
      to_start(k, s)
      ti_start(k + 2, s)
    return c

  lax.fori_loop(0, (kt - 4) // 2, t_pair, 0)
  for k, s in ((kt - 2, 0), (kt - 1, 1)):
    ti_wait(k, s)
    to_wait(k, s)
    t_cmp(s)
    to_start(k, s)
  to_wait(kt - 2, 0)
  to_wait(kt - 1, 1)

  plsc.subcore_barrier()

  pltpu.sync_copy(bt_hbm, bt_v)
  pltpu.sync_copy(x_hbm.at[wid], idx_all)
  bt_vecs = [[bt_v[r, 16 * d:16 * (d + 1)] for d in range(4)]
             for r in range(_R)]

  wr = [w_rows0, w_rows1]
  ar = [a_rows0, a_rows1]
  swg = [sem_w0, sem_w1]
  sag = [sem_a0, sem_a1]
  sou = [sem_o0, sem_o1]
  at_mine = at_hbm.at[cid]

  def gw_refs(k, s):
    return (w_hbm.at[idx_all.at[k]], wr[s], swg[s])

  def ga_refs(k, s):
    return (at_mine.at[idx_all.at[k]], ar[s], sag[s])

  def go_refs(k, s):
    base = wid * n_w + k * _CHUNK
    return (wr[s], out_hbm.at[pl.ds(base, _CHUNK)], sou[s])

  def g_start(k, s):
    pltpu.async_copy(*gw_refs(k, s))
    pltpu.async_copy(*ga_refs(k, s))

  def g_wait(k, s):
    pltpu.make_async_copy(*gw_refs(k, s)).wait()
    pltpu.make_async_copy(*ga_refs(k, s)).wait()

  def o_start(k, s):
    pltpu.async_copy(*go_refs(k, s))

  def o_wait(k, s):
    pltpu.make_async_copy(*go_refs(k, s)).wait()

  def b_cmp(s):
    w_rows, a_rows = wr[s], ar[s]

    def per_index(i, c2):
      accs = [w_rows[i, 16 * d:16 * (d + 1)] for d in range(4)]
      a_vec = a_rows[i, :]
      for r in range(_R):
        sca = a_vec[r]
        for d in range(4):
          accs[d] = accs[d] + bt_vecs[r][d] * sca
      for d in range(4):
        w_rows[i, 16 * d:16 * (d + 1)] = accs[d]
      return c2

    lax.fori_loop(0, _CHUNK, per_index, 0)

  g_start(0, 0)
  g_start(1, 1)
  g_wait(0, 0)
  b_cmp(0)
  o_start(0, 0)
  g_start(2, 0)
  g_wait(1, 1)
  b_cmp(1)
  o_start(1, 1)
  g_start(3, 1)

  def b_pair(h, c):
    for off, s in ((2, 0), (3, 1)):
      k = 2 * h + off
      g_wait(k, s)
      o_wait(k, s)
      b_cmp(s)
      o_start(k, s)
      g_start(k + 2, s)
    return c

  lax.fori_loop(0, (g_total - 4) // 2, b_pair, 0)
  for k, s in ((g_total - 2, 0), (g_total - 1, 1)):
    g_wait(k, s)
    o_wait(k, s)
    b_cmp(s)
    o_start(k, s)
  o_wait(g_total - 2, 0)
  o_wait(g_total - 1, 1)


def kernel(x, weight, lora_a, lora_b):
  b, l = x.shape
  n = b * l
  v = weight.shape[0]
  g_total = n // (_NW * _CHUNK)
  xf = x.reshape(_NW, g_total, _CHUNK)
  a3 = lora_a.reshape(_R, v // _TW, _TW)
  bt = (lora_b * _SCALE).T

  mesh = plsc.VectorSubcoreMesh(core_axis_name="c", subcore_axis_name="s")
  f = pl.kernel(
      _sc_body,
      mesh=mesh,
      out_type=(
          jax.ShapeDtypeStruct((n, _D), jnp.float32),
          jax.ShapeDtypeStruct((_NC, v, _R), jnp.float32),
      ),
      scratch_types=[
          pltpu.VMEM((g_total, _CHUNK), jnp.int32),
          pltpu.VMEM((_CHUNK, _D), jnp.float32),
          pltpu.VMEM((_CHUNK, _D), jnp.float32),
          pltpu.VMEM((_CHUNK, _R), jnp.float32),
          pltpu.VMEM((_CHUNK, _R), jnp.float32),
          pltpu.VMEM((_R, _D), jnp.float32),
          pltpu.VMEM((_R, _TW), jnp.float32),
          pltpu.VMEM((_R, _TW), jnp.float32),
          pltpu.VMEM((_TW, _R), jnp.float32),
          pltpu.VMEM((_TW, _R), jnp.float32),
          pltpu.SemaphoreType.DMA,
          pltpu.SemaphoreType.DMA,
          pltpu.SemaphoreType.DMA,
          pltpu.SemaphoreType.DMA,
          pltpu.SemaphoreType.DMA,
          pltpu.SemaphoreType.DMA,
          pltpu.SemaphoreType.DMA,
          pltpu.SemaphoreType.DMA,
          pltpu.SemaphoreType.DMA,
          pltpu.SemaphoreType.DMA,
      ],
      compiler_params=pltpu.CompilerParams(use_tc_tiling_on_sc=False,
                                           needs_layout_passes=False),
  )
  out, _ = f(xf, weight, a3, bt)
  return out.reshape(b, l, _D)


# speedup vs baseline: 8.4717x; 1.5416x over previous
import jax
import jax.numpy as jnp
from jax import lax
from jax.experimental import pallas as pl
from jax.experimental.pallas import tpu as pltpu
from jax.experimental.pallas import tpu_sc as plsc

_NC = 2
_NS = 16
_NW = _NC * _NS
_D = 64
_R = 16
_SCALE = 2.0
_CHUNK = 128
_TW = 250


def _sc_body(x_hbm, w_hbm, a_hbm, bt_hbm, out_hbm, at_hbm,
             idx_all, w_rows0, w_rows1, a_rows0, a_rows1, bt_v,
             tbuf0, tbuf1, tout0, tout1,
             sem_w0, sem_w1, sem_a0, sem_a1, sem_o0, sem_o1,
             sem_ti0, sem_ti1, sem_to0, sem_to1):
  cid = lax.axis_index("c")
  tid = lax.axis_index("s")
  wid = tid * _NC + cid
  kt = a_hbm.shape[1] // _NS
  j0 = tid * kt
  g_total = idx_all.shape[0]
  n_w = g_total * _CHUNK
  lane_iota = lax.iota(jnp.int32, 16)

  tb = [tbuf0, tbuf1]
  to = [tout0, tout1]
  sti = [sem_ti0, sem_ti1]
  sto = [sem_to0, sem_to1]

  def ti_refs(k, s):
    return (a_hbm.at[:, j0 + k, :], tb[s], sti[s])

  def to_refs(k, s):
    return (to[s], at_hbm.at[cid, pl.ds((j0 + k) * _TW, _TW), :], sto[s])

  def ti_start(k, s):
    pltpu.async_copy(*ti_refs(k, s))

  def ti_wait(k, s):
    pltpu.make_async_copy(*ti_refs(k, s)).wait()

  def to_start(k, s):
    pltpu.async_copy(*to_refs(k, s))

  def to_wait(k, s):
    pltpu.make_async_copy(*to_refs(k, s)).wait()

  def t_cmp(s):
    tbuf, tout = tb[s], to[s]

    @plsc.parallel_loop(0, _TW, unroll=4)
    def body(i):
      vec = plsc.load_gather(tbuf, [lane_iota, jnp.full((16,), i, jnp.int32)])
      tout[i, :] = vec

  ti_start(0, 0)
  ti_start(1, 1)
  ti_wait(0, 0)
  t_cmp(0)
  to_start(0, 0)
  ti_start(2, 0)
  ti_wait(1, 1)
  t_cmp(1)
  to_start(1, 1)
  ti_start(3, 1)

  def t_pair(h, c):
    for off, s in ((2, 0), (3, 1)):
      k = 2 * h + off
      ti_wait(k, s)
      to_wait(k, s)
      t_cmp(s)
      to_start(k, s)
      ti_start(k + 2, s)
    return c

  lax.fori_loop(0, (kt - 4) // 2, t_pair, 0)
  for k, s in ((kt - 2, 0), (kt - 1, 1)):
    ti_wait(k, s)
    to_wait(k, s)
    t_cmp(s)
    to_start(k, s)
  to_wait(kt - 2, 0)
  to_wait(kt - 1, 1)

  plsc.subcore_barrier()

  pltpu.sync_copy(bt_hbm, bt_v)
  pltpu.sync_copy(x_hbm.at[wid], idx_all)
  bt_bf = []
  for r in range(_R):
    row = [bt_v[r, 16 * d:16 * (d + 1)] for d in range(4)]
    bt_bf.append((
        plsc.pack(row[0], row[1], format=plsc.PackFormat.INTERLEAVED),
        plsc.pack(row[2], row[3], format=plsc.PackFormat.INTERLEAVED),
    ))

  wr = [w_rows0, w_rows1]
  ar = [a_rows0, a_rows1]
  swg = [sem_w0, sem_w1]
  sag = [sem_a0, sem_a1]
  sou = [sem_o0, sem_o1]
  at_mine = at_hbm.at[cid]

  def gw_refs(k, s):
    return (w_hbm.at[idx_all.at[k]], wr[s], swg[s])

  def ga_refs(k, s):
    return (at_mine.at[idx_all.at[k]], ar[s], sag[s])

  def go_refs(k, s):
    base = wid * n_w + k * _CHUNK
    return (wr[s], out_hbm.at[pl.ds(base, _CHUNK)], sou[s])

  def g_start(k, s):
    pltpu.async_copy(*gw_refs(k, s))
    pltpu.async_copy(*ga_refs(k, s))

  def g_wait(k, s):
    pltpu.make_async_copy(*gw_refs(k, s)).wait()
    pltpu.make_async_copy(*ga_refs(k, s)).wait()

  def o_start(k, s):
    pltpu.async_copy(*go_refs(k, s))

  def o_wait(k, s):
    pltpu.make_async_copy(*go_refs(k, s)).wait()

  def b_cmp(s):
    w_rows, a_rows = wr[s], ar[s]

    @plsc.parallel_loop(0, _CHUNK, unroll=2)
    def per_index(i):
      a_vec = a_rows[i, :]
      acc0 = jnp.zeros((32,), jnp.bfloat16)
      acc1 = jnp.zeros((32,), jnp.bfloat16)
      for r in range(_R):
        s_f = jnp.broadcast_to(a_vec[r], (16,))
        s_bf = plsc.pack(s_f, s_f, format=plsc.PackFormat.INTERLEAVED)
        acc0 = acc0 + bt_bf[r][0] * s_bf
        acc1 = acc1 + bt_bf[r][1] * s_bf
      d0, d1 = plsc.unpack(acc0, format=plsc.PackFormat.INTERLEAVED)
      d2, d3 = plsc.unpack(acc1, format=plsc.PackFormat.INTERLEAVED)
      for d, dv in enumerate((d0, d1, d2, d3)):
        w_rows[i, 16 * d:16 * (d + 1)] = w_rows[i, 16 * d:16 * (d + 1)] + dv

  g_start(0, 0)
  g_start(1, 1)
  g_wait(0, 0)
  b_cmp(0)
  o_start(0, 0)
  g_start(2, 0)
  g_wait(1, 1)
  b_cmp(1)
  o_start(1, 1)
  g_start(3, 1)

  def b_pair(h, c):
    for off, s in ((2, 0), (3, 1)):
      k = 2 * h + off
      g_wait(k, s)
      o_wait(k, s)
      b_cmp(s)
      o_start(k, s)
      g_start(k + 2, s)
    return c

  lax.fori_loop(0, (g_total - 4) // 2, b_pair, 0)
  for k, s in ((g_total - 2, 0), (g_total - 1, 1)):
    g_wait(k, s)
    o_wait(k, s)
    b_cmp(s)
    o_start(k, s)
  o_wait(g_total - 2, 0)
  o_wait(g_total - 1, 1)


def kernel(x, weight, lora_a, lora_b):
  b, l = x.shape
  n = b * l
  v = weight.shape[0]
  g_total = n // (_NW * _CHUNK)
  xf = x.reshape(_NW, g_total, _CHUNK)
  a3 = lora_a.reshape(_R, v // _TW, _TW)
  bt = (lora_b * _SCALE).T

  mesh = plsc.VectorSubcoreMesh(core_axis_name="c", subcore_axis_name="s")
  f = pl.kernel(
      _sc_body,
      mesh=mesh,
      out_type=(
          jax.ShapeDtypeStruct((n, _D), jnp.float32),
          jax.ShapeDtypeStruct((_NC, v, _R), jnp.float32),
      ),
      scratch_types=[
          pltpu.VMEM((g_total, _CHUNK), jnp.int32),
          pltpu.VMEM((_CHUNK, _D), jnp.float32),
          pltpu.VMEM((_CHUNK, _D), jnp.float32),
          pltpu.VMEM((_CHUNK, _R), jnp.float32),
          pltpu.VMEM((_CHUNK, _R), jnp.float32),
          pltpu.VMEM((_R, _D), jnp.float32),
          pltpu.VMEM((_R, _TW), jnp.float32),
          pltpu.VMEM((_R, _TW), jnp.float32),
          pltpu.VMEM((_TW, _R), jnp.float32),
          pltpu.VMEM((_TW, _R), jnp.float32),
          pltpu.SemaphoreType.DMA,
          pltpu.SemaphoreType.DMA,
          pltpu.SemaphoreType.DMA,
          pltpu.SemaphoreType.DMA,
          pltpu.SemaphoreType.DMA,
          pltpu.SemaphoreType.DMA,
          pltpu.SemaphoreType.DMA,
          pltpu.SemaphoreType.DMA,
          pltpu.SemaphoreType.DMA,
          pltpu.SemaphoreType.DMA,
      ],
      compiler_params=pltpu.CompilerParams(use_tc_tiling_on_sc=False,
                                           needs_layout_passes=False),
  )
  out, _ = f(xf, weight, a3, bt)
  return out.reshape(b, l, _D)
